# baseline (speedup 1.0000x reference)
import jax
import jax.numpy as jnp
from jax.experimental import pallas as pl
from jax.experimental.pallas import tpu as pltpu

_DIM = 384
_QK = 32
_PD = 96
_N = 1024
_B = 8
_EPS = 1e-5
_SCALE = _QK ** (-0.5)
_HI = jax.lax.Precision.HIGHEST
_MED = jax.lax.Precision.HIGHEST


def _gate_body(x_ref, w1t_ref, b1_ref, w2_ref, b2_ref, out_ref, acc_ref):
    b = pl.program_id(0)

    @pl.when(b == 0)
    def _init():
        acc_ref[0] = jnp.float32(0.0)

    xb = x_ref[0]
    g1 = jnp.dot(xb, w1t_ref[...], preferred_element_type=jnp.float32,
                 precision=_HI)
    g1 = jnp.maximum(g1 + b1_ref[...], 0.0)
    z = jnp.dot(g1, w2_ref[...], preferred_element_type=jnp.float32,
                precision=_HI) + b2_ref[0]
    acc_ref[0] = acc_ref[0] + jnp.sum(jax.nn.sigmoid(z))

    @pl.when(b == _B - 1)
    def _fin():
        gm = acc_ref[0] / jnp.float32(_B * _N)
        gm = jnp.where(jnp.isnan(gm), jnp.float32(0.5), gm)
        out_ref[0] = jnp.clip(
            jnp.floor(jnp.float32(_N) * gm).astype(jnp.int32), 1, _N)


def _main_body(kd_ref, x1_ref, x2_ref, gnw_ref, gnb_ref,
               wq_ref, bq_ref, wk_ref, bk_ref, wv_ref, bv_ref,
               wp1_ref, wp2_ref, bp_ref, out_ref):
    x1 = x1_ref[0]
    x2 = x2_ref[0]

    mu = jnp.mean(x1)
    var = jnp.mean((x1 - mu) ** 2)
    xn = (x1 - mu) * jax.lax.rsqrt(var + _EPS)
    xn = xn * gnw_ref[...] + gnb_ref[...]

    q = jnp.dot(xn, wq_ref[...], preferred_element_type=jnp.float32,
                precision=_HI) + bq_ref[...]
    k = jnp.dot(xn, wk_ref[...], preferred_element_type=jnp.float32,
                precision=_HI) + bk_ref[...]
    v = jnp.dot(xn, wv_ref[...], preferred_element_type=jnp.float32,
                precision=_HI) + bv_ref[...]

    attn = jax.lax.dot_general(
        q, k, (((1,), (1,)), ((), ())),
        preferred_element_type=jnp.float32,
        precision=_HI) * jnp.float32(_SCALE)

    kd = kd_ref[0]

    u = jax.lax.bitcast_convert_type(attn, jnp.uint32)
    uk = jnp.where(u >= jnp.uint32(0x80000000), ~u,
                   u | jnp.uint32(0x80000000))

    ones_col = jnp.ones((_N, 1), jnp.bfloat16)
    kdf = kd.astype(jnp.float32)

    def body(i, prefix):
        bit = (31 - i).astype(jnp.uint32)
        cand = prefix | (jnp.uint32(1) << bit)
        ge = (uk >= cand).astype(jnp.bfloat16)
        cnt = jnp.dot(ge, ones_col,
                      preferred_element_type=jnp.float32)
        return jnp.where(cnt >= kdf, cand, prefix)

    theta = jax.lax.fori_loop(0, 32, body,
                              jnp.zeros((_N, 1), jnp.uint32))
    maskf = (uk >= theta).astype(jnp.float32)

    m = jnp.max(attn, axis=1, keepdims=True)
    e = jnp.exp(attn - m) * maskf
    p = e / jnp.sum(e, axis=1, keepdims=True)

    o1 = jnp.dot(p, v, preferred_element_type=jnp.float32,
                 precision=_HI)
    s1 = o1 * jax.nn.sigmoid(o1)
    s2 = x2 * jax.nn.sigmoid(x2)
    y = (jnp.dot(s1, wp1_ref[...], preferred_element_type=jnp.float32,
                 precision=_HI)
         + jnp.dot(s2, wp2_ref[...], preferred_element_type=jnp.float32,
                   precision=_HI)
         + bp_ref[...])
    out_ref[0] = y


def kernel(x, gn_w, gn_b, W_qkv, bn_qkv_g, bn_qkv_b, W_proj, bn_proj_g,
           bn_proj_b, Wg1, bg1, Wg2, bg2):
    Bs, C, Hh, Ww = x.shape
    N = Hh * Ww

    xt = jnp.transpose(x.reshape(Bs, C, N), (0, 2, 1))
    x1t = xt[:, :, :_PD]
    x2t = xt[:, :, _PD:]

    bnq_s = bn_qkv_g / jnp.sqrt(1.0 + _EPS)
    Wqkv_eff = W_qkv * bnq_s[:, None]
    WqT = Wqkv_eff[:_QK].T
    WkT = Wqkv_eff[_QK:2 * _QK].T
    WvT = Wqkv_eff[2 * _QK:].T
    bq = bn_qkv_b[None, :_QK]
    bk = bn_qkv_b[None, _QK:2 * _QK]
    bv = bn_qkv_b[None, 2 * _QK:]

    bnp_s = bn_proj_g / jnp.sqrt(1.0 + _EPS)
    Wproj_eff = (W_proj * bnp_s[:, None]).T
    Wp1 = Wproj_eff[:_PD]
    Wp2 = Wproj_eff[_PD:]
    bp = bn_proj_b[None, :]

    Wg1T = Wg1.T
    bg1r = bg1[None, :]
    Wg2T = Wg2.T

    kd = pl.pallas_call(
        _gate_body,
        grid=(Bs,),
        in_specs=[
            pl.BlockSpec((1, N, C), lambda b: (b, 0, 0)),
            pl.BlockSpec((C, C // 2), lambda b: (0, 0)),
            pl.BlockSpec((1, C // 2), lambda b: (0, 0)),
            pl.BlockSpec((C // 2, 1), lambda b: (0, 0)),
            pl.BlockSpec(memory_space=pltpu.SMEM),
        ],
        out_specs=pl.BlockSpec(memory_space=pltpu.SMEM),
        out_shape=jax.ShapeDtypeStruct((1,), jnp.int32),
        scratch_shapes=[pltpu.SMEM((1,), jnp.float32)],
    )(xt, Wg1T, bg1r, Wg2T, bg2)

    grid_spec = pltpu.PrefetchScalarGridSpec(
        num_scalar_prefetch=1,
        grid=(Bs,),
        in_specs=[
            pl.BlockSpec((1, N, _PD), lambda b, kd: (b, 0, 0)),
            pl.BlockSpec((1, N, C - _PD), lambda b, kd: (b, 0, 0)),
            pl.BlockSpec((1, _PD), lambda b, kd: (0, 0)),
            pl.BlockSpec((1, _PD), lambda b, kd: (0, 0)),
            pl.BlockSpec((_PD, _QK), lambda b, kd: (0, 0)),
            pl.BlockSpec((1, _QK), lambda b, kd: (0, 0)),
            pl.BlockSpec((_PD, _QK), lambda b, kd: (0, 0)),
            pl.BlockSpec((1, _QK), lambda b, kd: (0, 0)),
            pl.BlockSpec((_PD, _PD), lambda b, kd: (0, 0)),
            pl.BlockSpec((1, _PD), lambda b, kd: (0, 0)),
            pl.BlockSpec((_PD, C), lambda b, kd: (0, 0)),
            pl.BlockSpec((C - _PD, C), lambda b, kd: (0, 0)),
            pl.BlockSpec((1, C), lambda b, kd: (0, 0)),
        ],
        out_specs=pl.BlockSpec((1, N, C), lambda b, kd: (b, 0, 0)),
    )
    yt = pl.pallas_call(
        _main_body,
        grid_spec=grid_spec,
        out_shape=jax.ShapeDtypeStruct((Bs, N, C), jnp.float32),
    )(kd, x1t, x2t, gn_w[None, :], gn_b[None, :],
      WqT, bq, WkT, bk, WvT, bv, Wp1, Wp2, bp)

    return jnp.transpose(yt, (0, 2, 1)).reshape(Bs, C, Hh, Ww)


# speedup vs baseline: 1.3296x; 1.3296x over previous
import jax
import jax.numpy as jnp
from jax.experimental import pallas as pl
from jax.experimental.pallas import tpu as pltpu

_DIM = 384
_QK = 32
_PD = 96
_N = 1024
_B = 8
_EPS = 1e-5
_SCALE = _QK ** (-0.5)
_HI = jax.lax.Precision.HIGHEST
_MED = jax.lax.Precision.HIGHEST


def _gate_body(x_ref, w1t_ref, b1_ref, w2_ref, b2_ref, out_ref, acc_ref):
    b = pl.program_id(0)

    @pl.when(b == 0)
    def _init():
        acc_ref[0] = jnp.float32(0.0)

    xb = x_ref[0]
    g1 = jnp.dot(xb, w1t_ref[...], preferred_element_type=jnp.float32,
                 precision=_HI)
    g1 = jnp.maximum(g1 + b1_ref[...], 0.0)
    z = jnp.dot(g1, w2_ref[...], preferred_element_type=jnp.float32,
                precision=_HI) + b2_ref[0]
    acc_ref[0] = acc_ref[0] + jnp.sum(jax.nn.sigmoid(z))

    @pl.when(b == _B - 1)
    def _fin():
        gm = acc_ref[0] / jnp.float32(_B * _N)
        gm = jnp.where(jnp.isnan(gm), jnp.float32(0.5), gm)
        out_ref[0] = jnp.clip(
            jnp.floor(jnp.float32(_N) * gm).astype(jnp.int32), 1, _N)


def _main_body(kd_ref, x1_ref, x2_ref, gnw_ref, gnb_ref,
               wq_ref, bq_ref, wk_ref, bk_ref, wv_ref, bv_ref,
               wp1_ref, wp2_ref, bp_ref, out_ref):
    x1 = x1_ref[0]
    x2 = x2_ref[0]

    mu = jnp.mean(x1)
    var = jnp.mean((x1 - mu) ** 2)
    xn = (x1 - mu) * jax.lax.rsqrt(var + _EPS)
    xn = xn * gnw_ref[...] + gnb_ref[...]

    q = jnp.dot(xn, wq_ref[...], preferred_element_type=jnp.float32,
                precision=_HI) + bq_ref[...]
    k = jnp.dot(xn, wk_ref[...], preferred_element_type=jnp.float32,
                precision=_HI) + bk_ref[...]
    v = jnp.dot(xn, wv_ref[...], preferred_element_type=jnp.float32,
                precision=_HI) + bv_ref[...]

    attn_t = jax.lax.dot_general(
        k, q, (((1,), (1,)), ((), ())),
        preferred_element_type=jnp.float32,
        precision=_HI) * jnp.float32(_SCALE)

    kd = kd_ref[0]

    u = jax.lax.bitcast_convert_type(attn_t, jnp.uint32)
    uk = jnp.where(u >= jnp.uint32(0x80000000), ~u,
                   u | jnp.uint32(0x80000000))

    def body(i, prefix):
        bit = (31 - i).astype(jnp.uint32)
        cand = prefix | (jnp.uint32(1) << bit)
        cnt = jnp.sum((uk >= cand).astype(jnp.int32), axis=0,
                      keepdims=True)
        return jnp.where(cnt >= kd, cand, prefix)

    theta = jax.lax.fori_loop(0, 32, body,
                              jnp.zeros((1, _N), jnp.uint32))
    maskf = (uk >= theta).astype(jnp.float32)

    m = jnp.max(attn_t, axis=0, keepdims=True)
    e = jnp.exp(attn_t - m) * maskf
    p = e / jnp.sum(e, axis=0, keepdims=True)

    o1 = jax.lax.dot_general(
        p, v, (((0,), (0,)), ((), ())),
        preferred_element_type=jnp.float32,
        precision=_HI)
    s1 = o1 * jax.nn.sigmoid(o1)
    s2 = x2 * jax.nn.sigmoid(x2)
    y = (jnp.dot(s1, wp1_ref[...], preferred_element_type=jnp.float32,
                 precision=_HI)
         + jnp.dot(s2, wp2_ref[...], preferred_element_type=jnp.float32,
                   precision=_HI)
         + bp_ref[...])
    out_ref[0] = y


def kernel(x, gn_w, gn_b, W_qkv, bn_qkv_g, bn_qkv_b, W_proj, bn_proj_g,
           bn_proj_b, Wg1, bg1, Wg2, bg2):
    Bs, C, Hh, Ww = x.shape
    N = Hh * Ww

    xt = jnp.transpose(x.reshape(Bs, C, N), (0, 2, 1))
    x1t = xt[:, :, :_PD]
    x2t = xt[:, :, _PD:]

    bnq_s = bn_qkv_g / jnp.sqrt(1.0 + _EPS)
    Wqkv_eff = W_qkv * bnq_s[:, None]
    WqT = Wqkv_eff[:_QK].T
    WkT = Wqkv_eff[_QK:2 * _QK].T
    WvT = Wqkv_eff[2 * _QK:].T
    bq = bn_qkv_b[None, :_QK]
    bk = bn_qkv_b[None, _QK:2 * _QK]
    bv = bn_qkv_b[None, 2 * _QK:]

    bnp_s = bn_proj_g / jnp.sqrt(1.0 + _EPS)
    Wproj_eff = (W_proj * bnp_s[:, None]).T
    Wp1 = Wproj_eff[:_PD]
    Wp2 = Wproj_eff[_PD:]
    bp = bn_proj_b[None, :]

    Wg1T = Wg1.T
    bg1r = bg1[None, :]
    Wg2T = Wg2.T

    kd = pl.pallas_call(
        _gate_body,
        grid=(Bs,),
        in_specs=[
            pl.BlockSpec((1, N, C), lambda b: (b, 0, 0)),
            pl.BlockSpec((C, C // 2), lambda b: (0, 0)),
            pl.BlockSpec((1, C // 2), lambda b: (0, 0)),
            pl.BlockSpec((C // 2, 1), lambda b: (0, 0)),
            pl.BlockSpec(memory_space=pltpu.SMEM),
        ],
        out_specs=pl.BlockSpec(memory_space=pltpu.SMEM),
        out_shape=jax.ShapeDtypeStruct((1,), jnp.int32),
        scratch_shapes=[pltpu.SMEM((1,), jnp.float32)],
    )(xt, Wg1T, bg1r, Wg2T, bg2)

    grid_spec = pltpu.PrefetchScalarGridSpec(
        num_scalar_prefetch=1,
        grid=(Bs,),
        in_specs=[
            pl.BlockSpec((1, N, _PD), lambda b, kd: (b, 0, 0)),
            pl.BlockSpec((1, N, C - _PD), lambda b, kd: (b, 0, 0)),
            pl.BlockSpec((1, _PD), lambda b, kd: (0, 0)),
            pl.BlockSpec((1, _PD), lambda b, kd: (0, 0)),
            pl.BlockSpec((_PD, _QK), lambda b, kd: (0, 0)),
            pl.BlockSpec((1, _QK), lambda b, kd: (0, 0)),
            pl.BlockSpec((_PD, _QK), lambda b, kd: (0, 0)),
            pl.BlockSpec((1, _QK), lambda b, kd: (0, 0)),
            pl.BlockSpec((_PD, _PD), lambda b, kd: (0, 0)),
            pl.BlockSpec((1, _PD), lambda b, kd: (0, 0)),
            pl.BlockSpec((_PD, C), lambda b, kd: (0, 0)),
            pl.BlockSpec((C - _PD, C), lambda b, kd: (0, 0)),
            pl.BlockSpec((1, C), lambda b, kd: (0, 0)),
        ],
        out_specs=pl.BlockSpec((1, N, C), lambda b, kd: (b, 0, 0)),
    )
    yt = pl.pallas_call(
        _main_body,
        grid_spec=grid_spec,
        out_shape=jax.ShapeDtypeStruct((Bs, N, C), jnp.float32),
    )(kd, x1t, x2t, gn_w[None, :], gn_b[None, :],
      WqT, bq, WkT, bk, WvT, bv, Wp1, Wp2, bp)

    return jnp.transpose(yt, (0, 2, 1)).reshape(Bs, C, Hh, Ww)


# speedup vs baseline: 1.7511x; 1.3171x over previous
import jax
import jax.numpy as jnp
from jax.experimental import pallas as pl
from jax.experimental.pallas import tpu as pltpu

_DIM = 384
_QK = 32
_PD = 96
_N = 1024
_B = 8
_EPS = 1e-5
_SCALE = _QK ** (-0.5)


def _dot3(a, b, dims=(((1,), (0,)), ((), ()))):
    ah = a.astype(jnp.bfloat16)
    al = (a - ah.astype(jnp.float32)).astype(jnp.bfloat16)
    bh = b.astype(jnp.bfloat16)
    bl = (b - bh.astype(jnp.float32)).astype(jnp.bfloat16)

    def d(x, y):
        return jax.lax.dot_general(x, y, dims,
                                   preferred_element_type=jnp.float32)

    return d(ah, bh) + d(ah, bl) + d(al, bh)


def _gate_body(x_ref, w1_ref, b1_ref, w2_ref, b2_ref, out_ref, acc_ref):
    b = pl.program_id(0)

    @pl.when(b == 0)
    def _init():
        acc_ref[...] = jnp.zeros_like(acc_ref)

    xb = x_ref[0]
    g1 = jnp.maximum(_dot3(w1_ref[...], xb) + b1_ref[...], 0.0)
    z = _dot3(w2_ref[...], g1) + b2_ref[0]
    acc_ref[...] = acc_ref[...] + jax.nn.sigmoid(z)

    @pl.when(b == _B - 1)
    def _fin():
        gm = jnp.sum(acc_ref[...]) / jnp.float32(_B * _N)
        gm = jnp.where(jnp.isnan(gm), jnp.float32(0.5), gm)
        out_ref[0] = jnp.clip(
            jnp.floor(jnp.float32(_N) * gm).astype(jnp.int32), 1, _N)


def _main_body(kd_ref, x_ref, gnw_ref, gnb_ref,
               wq_ref, bq_ref, wk_ref, bk_ref, wv_ref, bv_ref,
               wp1_ref, wp2_ref, bp_ref, out_ref):
    xb = x_ref[0]
    x1 = xb[:_PD]
    x2 = xb[_PD:]

    mu = jnp.mean(x1)
    var = jnp.mean((x1 - mu) ** 2)
    xn = (x1 - mu) * jax.lax.rsqrt(var + _EPS)
    xn = xn * gnw_ref[...] + gnb_ref[...]

    q = _dot3(wq_ref[...], xn) + bq_ref[...]
    k = _dot3(wk_ref[...], xn) + bk_ref[...]
    v = _dot3(wv_ref[...], xn) + bv_ref[...]

    attn_t = _dot3(k, q, (((0,), (0,)), ((), ()))) \
        * jnp.float32(_SCALE)

    kd = kd_ref[0]

    u = jax.lax.bitcast_convert_type(attn_t, jnp.uint32)
    uk = jnp.where(u >= jnp.uint32(0x80000000), ~u,
                   u | jnp.uint32(0x80000000))

    def body(i, prefix):
        bit = (31 - i).astype(jnp.uint32)
        cand = prefix | (jnp.uint32(1) << bit)
        cnt = jnp.sum((uk >= cand).astype(jnp.int32), axis=0,
                      keepdims=True)
        return jnp.where(cnt >= kd, cand, prefix)

    theta = jax.lax.fori_loop(0, 32, body,
                              jnp.zeros((1, _N), jnp.uint32))
    maskf = (uk >= theta).astype(jnp.float32)

    m = jnp.max(attn_t, axis=0, keepdims=True)
    e = jnp.exp(attn_t - m) * maskf
    p = e / jnp.sum(e, axis=0, keepdims=True)

    o1 = _dot3(v, p)
    s1 = o1 * jax.nn.sigmoid(o1)
    s2 = x2 * jax.nn.sigmoid(x2)
    y = (_dot3(wp1_ref[...], s1) + _dot3(wp2_ref[...], s2)
         + bp_ref[...])
    out_ref[0] = y


def kernel(x, gn_w, gn_b, W_qkv, bn_qkv_g, bn_qkv_b, W_proj, bn_proj_g,
           bn_proj_b, Wg1, bg1, Wg2, bg2):
    Bs, C, Hh, Ww = x.shape
    N = Hh * Ww

    xc = x.reshape(Bs, C, N)

    bnq_s = bn_qkv_g / jnp.sqrt(1.0 + _EPS)
    Wqkv_eff = W_qkv * bnq_s[:, None]
    Wq = Wqkv_eff[:_QK]
    Wk = Wqkv_eff[_QK:2 * _QK]
    Wv = Wqkv_eff[2 * _QK:]
    bq = bn_qkv_b[:_QK, None]
    bk = bn_qkv_b[_QK:2 * _QK, None]
    bv = bn_qkv_b[2 * _QK:, None]

    bnp_s = bn_proj_g / jnp.sqrt(1.0 + _EPS)
    Wproj_eff = W_proj * bnp_s[:, None]
    Wp1 = Wproj_eff[:, :_PD]
    Wp2 = Wproj_eff[:, _PD:]
    bp = bn_proj_b[:, None]

    kd = pl.pallas_call(
        _gate_body,
        grid=(Bs,),
        in_specs=[
            pl.BlockSpec((1, C, N), lambda b: (b, 0, 0)),
            pl.BlockSpec((C // 2, C), lambda b: (0, 0)),
            pl.BlockSpec((C // 2, 1), lambda b: (0, 0)),
            pl.BlockSpec((1, C // 2), lambda b: (0, 0)),
            pl.BlockSpec(memory_space=pltpu.SMEM),
        ],
        out_specs=pl.BlockSpec(memory_space=pltpu.SMEM),
        out_shape=jax.ShapeDtypeStruct((1,), jnp.int32),
        scratch_shapes=[pltpu.VMEM((1, N), jnp.float32)],
    )(xc, Wg1, bg1[:, None], Wg2, bg2)

    grid_spec = pltpu.PrefetchScalarGridSpec(
        num_scalar_prefetch=1,
        grid=(Bs,),
        in_specs=[
            pl.BlockSpec((1, C, N), lambda b, kd: (b, 0, 0)),
            pl.BlockSpec((_PD, 1), lambda b, kd: (0, 0)),
            pl.BlockSpec((_PD, 1), lambda b, kd: (0, 0)),
            pl.BlockSpec((_QK, _PD), lambda b, kd: (0, 0)),
            pl.BlockSpec((_QK, 1), lambda b, kd: (0, 0)),
            pl.BlockSpec((_QK, _PD), lambda b, kd: (0, 0)),
            pl.BlockSpec((_QK, 1), lambda b, kd: (0, 0)),
            pl.BlockSpec((_PD, _PD), lambda b, kd: (0, 0)),
            pl.BlockSpec((_PD, 1), lambda b, kd: (0, 0)),
            pl.BlockSpec((C, _PD), lambda b, kd: (0, 0)),
            pl.BlockSpec((C, C - _PD), lambda b, kd: (0, 0)),
            pl.BlockSpec((C, 1), lambda b, kd: (0, 0)),
        ],
        out_specs=pl.BlockSpec((1, C, N), lambda b, kd: (b, 0, 0)),
    )
    yc = pl.pallas_call(
        _main_body,
        grid_spec=grid_spec,
        out_shape=jax.ShapeDtypeStruct((Bs, C, N), jnp.float32),
    )(kd, xc, gn_w[:, None], gn_b[:, None],
      Wq, bq, Wk, bk, Wv, bv, Wp1, Wp2, bp)

    return yc.reshape(Bs, C, Hh, Ww)


# speedup vs baseline: 1.7581x; 1.0040x over previous
import jax
import jax.numpy as jnp
from jax.experimental import pallas as pl
from jax.experimental.pallas import tpu as pltpu

_DIM = 384
_QK = 32
_PD = 96
_N = 1024
_B = 8
_EPS = 1e-5
_SCALE = _QK ** (-0.5)


def _dot3(a, b, dims=(((1,), (0,)), ((), ()))):
    ah = a.astype(jnp.bfloat16)
    al = (a - ah.astype(jnp.float32)).astype(jnp.bfloat16)
    bh = b.astype(jnp.bfloat16)
    bl = (b - bh.astype(jnp.float32)).astype(jnp.bfloat16)

    def d(x, y):
        return jax.lax.dot_general(x, y, dims,
                                   preferred_element_type=jnp.float32)

    return d(ah, bh) + d(ah, bl) + d(al, bh)


def _gate_body(x_ref, w1_ref, b1_ref, w2_ref, b2_ref, out_ref, acc_ref):
    b = pl.program_id(0)

    @pl.when(b == 0)
    def _init():
        acc_ref[...] = jnp.zeros_like(acc_ref)

    xb = x_ref[0]
    g1 = jnp.maximum(_dot3(w1_ref[...], xb) + b1_ref[...], 0.0)
    z = _dot3(w2_ref[...], g1) + b2_ref[0]
    acc_ref[...] = acc_ref[...] + jax.nn.sigmoid(z)

    @pl.when(b == _B - 1)
    def _fin():
        gm = jnp.sum(acc_ref[...]) / jnp.float32(_B * _N)
        gm = jnp.where(jnp.isnan(gm), jnp.float32(0.5), gm)
        out_ref[0] = jnp.clip(
            jnp.floor(jnp.float32(_N) * gm).astype(jnp.int32), 1, _N)


def _main_body(kd_ref, x_ref, gnw_ref, gnb_ref,
               wq_ref, bq_ref, wk_ref, bk_ref, wv_ref, bv_ref,
               wp1_ref, wp2_ref, bp_ref, out_ref):
    xb = x_ref[0]
    x1 = xb[:_PD]
    x2 = xb[_PD:]

    mu = jnp.mean(x1)
    var = jnp.mean((x1 - mu) ** 2)
    xn = (x1 - mu) * jax.lax.rsqrt(var + _EPS)
    xn = xn * gnw_ref[...] + gnb_ref[...]

    q = _dot3(wq_ref[...], xn) + bq_ref[...]
    k = _dot3(wk_ref[...], xn) + bk_ref[...]
    v = _dot3(wv_ref[...], xn) + bv_ref[...]

    attn_t = _dot3(k, q, (((0,), (0,)), ((), ()))) \
        * jnp.float32(_SCALE)

    kd = kd_ref[0]

    u = jax.lax.bitcast_convert_type(attn_t, jnp.uint32)
    uk = jnp.where(u >= jnp.uint32(0x80000000), ~u,
                   u | jnp.uint32(0x80000000))

    def body(i, prefix):
        bit = (31 - i).astype(jnp.uint32)
        cand = prefix | (jnp.uint32(1) << bit)
        cnt = jnp.sum((uk >= cand).astype(jnp.int32), axis=0,
                      keepdims=True)
        return jnp.where(cnt >= kd, cand, prefix)

    theta = jax.lax.fori_loop(0, 32, body,
                              jnp.zeros((1, _N), jnp.uint32),
                              unroll=4)
    maskf = (uk >= theta).astype(jnp.float32)

    m = jnp.max(attn_t, axis=0, keepdims=True)
    e = jnp.exp(attn_t - m) * maskf
    p = e / jnp.sum(e, axis=0, keepdims=True)

    o1 = _dot3(v, p)
    s1 = o1 * jax.nn.sigmoid(o1)
    s2 = x2 * jax.nn.sigmoid(x2)
    y = (_dot3(wp1_ref[...], s1) + _dot3(wp2_ref[...], s2)
         + bp_ref[...])
    out_ref[0] = y


def kernel(x, gn_w, gn_b, W_qkv, bn_qkv_g, bn_qkv_b, W_proj, bn_proj_g,
           bn_proj_b, Wg1, bg1, Wg2, bg2):
    Bs, C, Hh, Ww = x.shape
    N = Hh * Ww

    xc = x.reshape(Bs, C, N)

    bnq_s = bn_qkv_g / jnp.sqrt(1.0 + _EPS)
    Wqkv_eff = W_qkv * bnq_s[:, None]
    Wq = Wqkv_eff[:_QK]
    Wk = Wqkv_eff[_QK:2 * _QK]
    Wv = Wqkv_eff[2 * _QK:]
    bq = bn_qkv_b[:_QK, None]
    bk = bn_qkv_b[_QK:2 * _QK, None]
    bv = bn_qkv_b[2 * _QK:, None]

    bnp_s = bn_proj_g / jnp.sqrt(1.0 + _EPS)
    Wproj_eff = W_proj * bnp_s[:, None]
    Wp1 = Wproj_eff[:, :_PD]
    Wp2 = Wproj_eff[:, _PD:]
    bp = bn_proj_b[:, None]

    kd = pl.pallas_call(
        _gate_body,
        grid=(Bs,),
        in_specs=[
            pl.BlockSpec((1, C, N), lambda b: (b, 0, 0)),
            pl.BlockSpec((C // 2, C), lambda b: (0, 0)),
            pl.BlockSpec((C // 2, 1), lambda b: (0, 0)),
            pl.BlockSpec((1, C // 2), lambda b: (0, 0)),
            pl.BlockSpec(memory_space=pltpu.SMEM),
        ],
        out_specs=pl.BlockSpec(memory_space=pltpu.SMEM),
        out_shape=jax.ShapeDtypeStruct((1,), jnp.int32),
        scratch_shapes=[pltpu.VMEM((1, N), jnp.float32)],
    )(xc, Wg1, bg1[:, None], Wg2, bg2)

    grid_spec = pltpu.PrefetchScalarGridSpec(
        num_scalar_prefetch=1,
        grid=(Bs,),
        in_specs=[
            pl.BlockSpec((1, C, N), lambda b, kd: (b, 0, 0)),
            pl.BlockSpec((_PD, 1), lambda b, kd: (0, 0)),
            pl.BlockSpec((_PD, 1), lambda b, kd: (0, 0)),
            pl.BlockSpec((_QK, _PD), lambda b, kd: (0, 0)),
            pl.BlockSpec((_QK, 1), lambda b, kd: (0, 0)),
            pl.BlockSpec((_QK, _PD), lambda b, kd: (0, 0)),
            pl.BlockSpec((_QK, 1), lambda b, kd: (0, 0)),
            pl.BlockSpec((_PD, _PD), lambda b, kd: (0, 0)),
            pl.BlockSpec((_PD, 1), lambda b, kd: (0, 0)),
            pl.BlockSpec((C, _PD), lambda b, kd: (0, 0)),
            pl.BlockSpec((C, C - _PD), lambda b, kd: (0, 0)),
            pl.BlockSpec((C, 1), lambda b, kd: (0, 0)),
        ],
        out_specs=pl.BlockSpec((1, C, N), lambda b, kd: (b, 0, 0)),
    )
    yc = pl.pallas_call(
        _main_body,
        grid_spec=grid_spec,
        out_shape=jax.ShapeDtypeStruct((Bs, C, N), jnp.float32),
    )(kd, xc, gn_w[:, None], gn_b[:, None],
      Wq, bq, Wk, bk, Wv, bv, Wp1, Wp2, bp)

    return yc.reshape(Bs, C, Hh, Ww)


# speedup vs baseline: 1.7671x; 1.0051x over previous
import jax
import jax.numpy as jnp
from jax.experimental import pallas as pl
from jax.experimental.pallas import tpu as pltpu

_DIM = 384
_QK = 32
_PD = 96
_N = 1024
_B = 8
_EPS = 1e-5
_SCALE = _QK ** (-0.5)


def _dot3(a, b, dims=(((1,), (0,)), ((), ()))):
    ah = a.astype(jnp.bfloat16)
    al = (a - ah.astype(jnp.float32)).astype(jnp.bfloat16)
    bh = b.astype(jnp.bfloat16)
    bl = (b - bh.astype(jnp.float32)).astype(jnp.bfloat16)

    def d(x, y):
        return jax.lax.dot_general(x, y, dims,
                                   preferred_element_type=jnp.float32)

    return d(ah, bh) + d(ah, bl) + d(al, bh)


def _body(x_ref, w1_ref, b1_ref, w2_ref, b2_ref, gnw_ref, gnb_ref,
          wq_ref, bq_ref, wk_ref, bk_ref, wv_ref, bv_ref,
          wp1_ref, wp2_ref, bp_ref, out_ref, acc_ref, kd_ref):
    s = pl.program_id(0)
    xb = x_ref[0]

    @pl.when(s == 0)
    def _init():
        acc_ref[...] = jnp.zeros_like(acc_ref)

    @pl.when(s < _B)
    def _gate():
        g1 = jnp.maximum(_dot3(w1_ref[...], xb) + b1_ref[...], 0.0)
        z = _dot3(w2_ref[...], g1) + b2_ref[0]
        acc_ref[...] = acc_ref[...] + jax.nn.sigmoid(z)

        @pl.when(s == _B - 1)
        def _fin():
            gm = jnp.sum(acc_ref[...]) / jnp.float32(_B * _N)
            gm = jnp.where(jnp.isnan(gm), jnp.float32(0.5), gm)
            kd_ref[0] = jnp.clip(
                jnp.floor(jnp.float32(_N) * gm).astype(jnp.int32), 1, _N)

    @pl.when(s >= _B)
    def _attn():
        x1 = xb[:_PD]
        x2 = xb[_PD:]

        mu = jnp.mean(x1)
        var = jnp.mean((x1 - mu) ** 2)
        xn = (x1 - mu) * jax.lax.rsqrt(var + _EPS)
        xn = xn * gnw_ref[...] + gnb_ref[...]

        q = _dot3(wq_ref[...], xn) + bq_ref[...]
        k = _dot3(wk_ref[...], xn) + bk_ref[...]
        v = _dot3(wv_ref[...], xn) + bv_ref[...]

        attn_t = _dot3(k, q, (((0,), (0,)), ((), ()))) \
            * jnp.float32(_SCALE)

        kd = kd_ref[0]

        u = jax.lax.bitcast_convert_type(attn_t, jnp.uint32)
        uk = jnp.where(u >= jnp.uint32(0x80000000), ~u,
                       u | jnp.uint32(0x80000000))

        def body(i, prefix):
            bit = (31 - i).astype(jnp.uint32)
            cand = prefix | (jnp.uint32(1) << bit)
            cnt = jnp.sum((uk >= cand).astype(jnp.int32), axis=0,
                          keepdims=True)
            return jnp.where(cnt >= kd, cand, prefix)

        theta = jax.lax.fori_loop(0, 32, body,
                                  jnp.zeros((1, _N), jnp.uint32),
                                  unroll=4)
        maskf = (uk >= theta).astype(jnp.float32)

        m = jnp.max(attn_t, axis=0, keepdims=True)
        e = jnp.exp(attn_t - m) * maskf
        p = e / jnp.sum(e, axis=0, keepdims=True)

        o1 = _dot3(v, p)
        s1 = o1 * jax.nn.sigmoid(o1)
        s2 = x2 * jax.nn.sigmoid(x2)
        y = (_dot3(wp1_ref[...], s1) + _dot3(wp2_ref[...], s2)
             + bp_ref[...])
        out_ref[0] = y


def kernel(x, gn_w, gn_b, W_qkv, bn_qkv_g, bn_qkv_b, W_proj, bn_proj_g,
           bn_proj_b, Wg1, bg1, Wg2, bg2):
    Bs, C, Hh, Ww = x.shape
    N = Hh * Ww

    xc = x.reshape(Bs, C, N)

    bnq_s = bn_qkv_g / jnp.sqrt(1.0 + _EPS)
    Wqkv_eff = W_qkv * bnq_s[:, None]
    Wq = Wqkv_eff[:_QK]
    Wk = Wqkv_eff[_QK:2 * _QK]
    Wv = Wqkv_eff[2 * _QK:]
    bq = bn_qkv_b[:_QK, None]
    bk = bn_qkv_b[_QK:2 * _QK, None]
    bv = bn_qkv_b[2 * _QK:, None]

    bnp_s = bn_proj_g / jnp.sqrt(1.0 + _EPS)
    Wproj_eff = W_proj * bnp_s[:, None]
    Wp1 = Wproj_eff[:, :_PD]
    Wp2 = Wproj_eff[:, _PD:]
    bp = bn_proj_b[:, None]

    def _w(shape):
        return pl.BlockSpec(shape, lambda s: tuple(0 for _ in shape))

    yc = pl.pallas_call(
        _body,
        grid=(2 * Bs,),
        in_specs=[
            pl.BlockSpec((1, C, N), lambda s: (s % Bs, 0, 0)),
            _w((C // 2, C)),
            _w((C // 2, 1)),
            _w((1, C // 2)),
            pl.BlockSpec(memory_space=pltpu.SMEM),
            _w((_PD, 1)),
            _w((_PD, 1)),
            _w((_QK, _PD)),
            _w((_QK, 1)),
            _w((_QK, _PD)),
            _w((_QK, 1)),
            _w((_PD, _PD)),
            _w((_PD, 1)),
            _w((C, _PD)),
            _w((C, C - _PD)),
            _w((C, 1)),
        ],
        out_specs=pl.BlockSpec(
            (1, C, N),
            lambda s: (jnp.where(s < Bs, 0, s - Bs), 0, 0)),
        out_shape=jax.ShapeDtypeStruct((Bs, C, N), jnp.float32),
        scratch_shapes=[pltpu.VMEM((1, N), jnp.float32),
                        pltpu.SMEM((1,), jnp.int32)],
    )(xc, Wg1, bg1[:, None], Wg2, bg2, gn_w[:, None], gn_b[:, None],
      Wq, bq, Wk, bk, Wv, bv, Wp1, Wp2, bp)

    return yc.reshape(Bs, C, Hh, Ww)
